# Initial kernel scaffold; baseline (speedup 1.0000x reference)
import jax
import jax.numpy as jnp
from jax.experimental import pallas as pl


def kernel(x, y, W_qkv, W_gate, b_gate, W_proj, b_proj):
    raise NotImplementedError("write your pallas kernel here")


# speedup vs baseline: 4.5706x; 4.5706x over previous
import functools

import jax
import jax.numpy as jnp
from jax.experimental import pallas as pl
from jax.experimental.pallas import tpu as pltpu

B = 4096
DIM = 1024
NUM_EXPERTS = 8
NUM_HEADS = 16
TOP_K = 2
HEAD_DIM = DIM // NUM_HEADS
SCALE = HEAD_DIM ** (-0.5)

BT = 512


def _routing_weights(scores):
    bt = scores.shape[0]
    e_iota = jax.lax.broadcasted_iota(jnp.int32, (bt, NUM_EXPERTS), 1)
    m1 = jnp.max(scores, axis=1, keepdims=True)
    idx1 = jnp.min(jnp.where(scores == m1, e_iota, NUM_EXPERTS), axis=1,
                   keepdims=True)
    masked = jnp.where(e_iota == idx1, -1.0, scores)
    m2 = jnp.max(masked, axis=1, keepdims=True)
    idx2 = jnp.min(jnp.where(masked == m2, e_iota, NUM_EXPERTS), axis=1,
                   keepdims=True)
    return jnp.where(e_iota == idx1, m1, 0.0) + jnp.where(e_iota == idx2, m2, 0.0)


def _attention(q, kv, wproj, bproj):
    bt = q.shape[0]
    q3 = q.reshape(bt, NUM_HEADS, HEAD_DIM)
    k3 = kv[:, :DIM].reshape(bt, NUM_HEADS, HEAD_DIM)
    v3 = kv[:, DIM:].reshape(bt, NUM_HEADS, HEAD_DIM)
    attn = jax.lax.dot_general(
        q3, k3, (((2,), (2,)), ((0,), (0,))),
        preferred_element_type=jnp.float32) * SCALE
    attn = attn - jnp.max(attn, axis=2, keepdims=True)
    attn = jnp.exp(attn)
    attn = attn / jnp.sum(attn, axis=2, keepdims=True)
    ctx = jax.lax.dot_general(
        attn, v3, (((2,), (1,)), ((0,), (0,))),
        preferred_element_type=jnp.float32)
    ctx = ctx.reshape(bt, DIM)
    return jnp.dot(ctx, wproj, preferred_element_type=jnp.float32) + bproj


def _moe_kernel(x_ref, y_ref, w_ref, wg_ref, bg_ref, wp_ref, bp_ref,
                out_ref, accq_ref, acckv_ref, gates_ref):
    e = pl.program_id(1)

    @pl.when(e == 0)
    def _():
        scores = jnp.dot(x_ref[...], wg_ref[...],
                         preferred_element_type=jnp.float32) + bg_ref[...]
        scores = scores - jnp.max(scores, axis=1, keepdims=True)
        scores = jnp.exp(scores)
        scores = scores / jnp.sum(scores, axis=1, keepdims=True)
        gates_ref[...] = _routing_weights(scores)
        accq_ref[...] = jnp.zeros_like(accq_ref)
        acckv_ref[...] = jnp.zeros_like(acckv_ref)

    gates = gates_ref[...]
    lane = jax.lax.broadcasted_iota(jnp.int32, gates.shape, 1)
    we = jnp.sum(jnp.where(lane == e, gates, 0.0), axis=1, keepdims=True)
    wq = w_ref[0, :, :DIM]
    wkv = w_ref[0, :, DIM:]
    accq_ref[...] += we * jnp.dot(y_ref[...], wq,
                                  preferred_element_type=jnp.float32)
    acckv_ref[...] += we * jnp.dot(x_ref[...], wkv,
                                   preferred_element_type=jnp.float32)

    @pl.when(e == NUM_EXPERTS - 1)
    def _():
        out_ref[...] = _attention(accq_ref[...], acckv_ref[...],
                                  wp_ref[...], bp_ref[...])


@jax.jit
def kernel(x, y, W_qkv, W_gate, b_gate, W_proj, b_proj):
    nt = B // BT
    grid = (nt, NUM_EXPERTS)
    W_proj_perm = (W_proj.reshape(HEAD_DIM, NUM_HEADS, DIM)
                   .transpose(1, 0, 2).reshape(DIM, DIM))
    out = pl.pallas_call(
        _moe_kernel,
        grid=grid,
        in_specs=[
            pl.BlockSpec((BT, DIM), lambda t, e: (t, 0)),
            pl.BlockSpec((BT, DIM), lambda t, e: (t, 0)),
            pl.BlockSpec((1, DIM, 3 * DIM), lambda t, e: (e, 0, 0)),
            pl.BlockSpec((DIM, NUM_EXPERTS), lambda t, e: (0, 0)),
            pl.BlockSpec((1, NUM_EXPERTS), lambda t, e: (0, 0)),
            pl.BlockSpec((DIM, DIM), lambda t, e: (0, 0)),
            pl.BlockSpec((1, DIM), lambda t, e: (0, 0)),
        ],
        out_specs=pl.BlockSpec((BT, DIM), lambda t, e: (t, 0)),
        out_shape=jax.ShapeDtypeStruct((B, DIM), jnp.float32),
        scratch_shapes=[
            pltpu.VMEM((BT, DIM), jnp.float32),
            pltpu.VMEM((BT, 2 * DIM), jnp.float32),
            pltpu.VMEM((BT, NUM_EXPERTS), jnp.float32),
        ],
        compiler_params=pltpu.CompilerParams(
            dimension_semantics=("arbitrary", "arbitrary"),
        ),
    )(x, y, W_qkv, W_gate, b_gate.reshape(1, NUM_EXPERTS),
      W_proj_perm, b_proj.reshape(1, DIM))
    return out
